# baseline (device time: 7876 ns/iter reference)
import os

import jax
import jax.numpy as jnp
from jax import lax
from jax.experimental import pallas as pl
from jax.experimental.pallas import tpu as pltpu

N_DEV = 8
_NO_RDMA = os.environ.get("KERNEL_NO_RDMA", "0") == "1"


def kernel(x, k):
    b, s, c = x.shape
    taps = k.shape[0]
    halo = taps - 1

    def body(x_hbm, k_hbm, out_hbm, x_ref, k_ref, out_ref, halo_ref,
             send_ref, load_sems, store_sems, send_sem, recv_sem):
        my_i = lax.axis_index("i")
        left = my_i - 1
        right = my_i + 1
        is_first = my_i == 0
        is_last = my_i == N_DEV - 1
        if _NO_RDMA:
            is_first = my_i >= 0
            is_last = my_i >= 0

        cp_x = pltpu.make_async_copy(x_hbm, x_ref, load_sems.at[0])
        cp_k = pltpu.make_async_copy(k_hbm, k_ref, load_sems.at[1])
        cp_x.start()
        cp_k.start()

        barrier_sem = pltpu.get_barrier_semaphore()

        @pl.when(jnp.logical_not(is_first))
        def _():
            pl.semaphore_signal(
                barrier_sem, inc=1,
                device_id=(left,), device_id_type=pl.DeviceIdType.MESH,
            )

        cp_x.wait()
        cp_k.wait()

        @pl.when(jnp.logical_not(is_last))
        def _():
            pl.semaphore_wait(barrier_sem, 1)
            send_ref[...] = x_ref[:, s - halo:, :]
            send = pltpu.make_async_remote_copy(
                src_ref=send_ref,
                dst_ref=halo_ref,
                send_sem=send_sem,
                recv_sem=recv_sem,
                device_id=(right,),
                device_id_type=pl.DeviceIdType.MESH,
            )
            send.start()

        kf = k_ref[...]
        xs = x_ref[...]
        acc = xs * kf[taps - 1].reshape(1, 1, c)
        for j in range(taps - 2, -1, -1):
            xs = pltpu.roll(xs, 1, 1)
            acc = acc + xs * kf[j].reshape(1, 1, c)
        out_ref[...] = (acc * (0.5 * jnp.tanh(0.5 * acc) + 0.5)).astype(
            jnp.bfloat16
        )

        st_bulk = pltpu.make_async_copy(
            out_ref.at[:, 8:, :], out_hbm.at[:, 8:, :], store_sems.at[0]
        )
        st_bulk.start()

        @pl.when(is_first)
        def _():
            halo_ref[...] = jnp.zeros((b, halo, c), x_ref.dtype)

        @pl.when(jnp.logical_not(is_first))
        def _():
            recv = pltpu.make_async_remote_copy(
                src_ref=send_ref,
                dst_ref=halo_ref,
                send_sem=send_sem,
                recv_sem=recv_sem,
                device_id=(left,),
                device_id_type=pl.DeviceIdType.MESH,
            )
            recv.wait_recv()

        pad3 = jnp.concatenate(
            [halo_ref[...], x_ref[:, :halo, :]], axis=1
        )
        eacc = jnp.zeros((b, halo, c), jnp.float32)
        for j in range(taps):
            kj = k_ref[j:j + 1, :].reshape(1, 1, c)
            eacc = eacc + pad3[:, j:j + halo, :] * kj
        out_ref[:, :halo, :] = (
            eacc * (0.5 * jnp.tanh(0.5 * eacc) + 0.5)
        ).astype(jnp.bfloat16)

        st_edge = pltpu.make_async_copy(
            out_ref.at[:, :8, :], out_hbm.at[:, :8, :], store_sems.at[1]
        )
        st_edge.start()

        st_bulk.wait()
        st_edge.wait()

        @pl.when(jnp.logical_not(is_last))
        def _():
            drain = pltpu.make_async_remote_copy(
                src_ref=send_ref,
                dst_ref=halo_ref,
                send_sem=send_sem,
                recv_sem=recv_sem,
                device_id=(right,),
                device_id_type=pl.DeviceIdType.MESH,
            )
            drain.wait_send()

    return pl.pallas_call(
        body,
        out_shape=jax.ShapeDtypeStruct((b, s, c), jnp.bfloat16),
        in_specs=[
            pl.BlockSpec(memory_space=pl.ANY),
            pl.BlockSpec(memory_space=pl.ANY),
        ],
        out_specs=pl.BlockSpec(memory_space=pl.ANY),
        scratch_shapes=[
            pltpu.VMEM((b, s, c), x.dtype),
            pltpu.VMEM((taps, c), k.dtype),
            pltpu.VMEM((b, s, c), jnp.bfloat16),
            pltpu.VMEM((b, halo, c), x.dtype),
            pltpu.VMEM((b, halo, c), x.dtype),
            pltpu.SemaphoreType.DMA((2,)),
            pltpu.SemaphoreType.DMA((2,)),
            pltpu.SemaphoreType.DMA,
            pltpu.SemaphoreType.DMA,
        ],
        compiler_params=pltpu.CompilerParams(collective_id=0),
    )(x, k)


# device time: 6483 ns/iter; 1.2149x vs baseline; 1.2149x over previous
import os

import jax
import jax.numpy as jnp
from jax import lax
from jax.experimental import pallas as pl
from jax.experimental.pallas import tpu as pltpu

N_DEV = 8
_NO_RDMA = os.environ.get("KERNEL_NO_RDMA", "0") == "1"


def kernel(x, k):
    b, s, c = x.shape
    taps = k.shape[0]
    halo = taps - 1

    def body(x_hbm, k_hbm, out_hbm, x_ref, k_ref, out_ref, halo_ref,
             send_ref, load_sems, store_sems, send_sem, recv_sem):
        my_i = lax.axis_index("i")
        left = my_i - 1
        right = my_i + 1
        is_first = my_i == 0
        is_last = my_i == N_DEV - 1
        if _NO_RDMA:
            is_first = my_i >= 0
            is_last = my_i >= 0

        cp_x = pltpu.make_async_copy(x_hbm, x_ref, load_sems.at[0])
        cp_k = pltpu.make_async_copy(k_hbm, k_ref, load_sems.at[1])
        cp_x.start()
        cp_k.start()

        barrier_sem = pltpu.get_barrier_semaphore()

        @pl.when(jnp.logical_not(is_first))
        def _():
            pl.semaphore_signal(
                barrier_sem, inc=1,
                device_id=(left,), device_id_type=pl.DeviceIdType.MESH,
            )

        cp_x.wait()
        cp_k.wait()

        send_ref[...] = x_ref[:, s - halo:, :]

        kf = k_ref[...]

        def conv_silu(xs):
            acc = xs * kf[taps - 1].reshape(1, 1, c)
            for j in range(taps - 2, -1, -1):
                xs = pltpu.roll(xs, 1, 1)
                acc = acc + xs * kf[j].reshape(1, 1, c)
            return (acc * (0.5 * jnp.tanh(0.5 * acc) + 0.5)).astype(
                jnp.bfloat16
            )

        half = s // 2
        out_ref[:, :half, :] = conv_silu(x_ref[:, :half, :])

        @pl.when(jnp.logical_not(is_last))
        def _():
            pl.semaphore_wait(barrier_sem, 1)
            send = pltpu.make_async_remote_copy(
                src_ref=send_ref,
                dst_ref=halo_ref,
                send_sem=send_sem,
                recv_sem=recv_sem,
                device_id=(right,),
                device_id_type=pl.DeviceIdType.MESH,
            )
            send.start()

        out_ref[:, half:, :] = conv_silu(x_ref[:, half - 8:, :])[:, 8:, :]

        st_bulk = pltpu.make_async_copy(
            out_ref.at[:, 8:, :], out_hbm.at[:, 8:, :], store_sems.at[0]
        )
        st_bulk.start()

        @pl.when(is_first)
        def _():
            halo_ref[...] = jnp.zeros((b, halo, c), x_ref.dtype)

        @pl.when(jnp.logical_not(is_first))
        def _():
            recv = pltpu.make_async_remote_copy(
                src_ref=send_ref,
                dst_ref=halo_ref,
                send_sem=send_sem,
                recv_sem=recv_sem,
                device_id=(left,),
                device_id_type=pl.DeviceIdType.MESH,
            )
            recv.wait_recv()

        pad3 = jnp.concatenate(
            [halo_ref[...], x_ref[:, :halo, :]], axis=1
        )
        eacc = jnp.zeros((b, halo, c), jnp.float32)
        for j in range(taps):
            kj = k_ref[j:j + 1, :].reshape(1, 1, c)
            eacc = eacc + pad3[:, j:j + halo, :] * kj
        out_ref[:, :halo, :] = (
            eacc * (0.5 * jnp.tanh(0.5 * eacc) + 0.5)
        ).astype(jnp.bfloat16)

        st_edge = pltpu.make_async_copy(
            out_ref.at[:, :8, :], out_hbm.at[:, :8, :], store_sems.at[1]
        )
        st_edge.start()

        st_bulk.wait()
        st_edge.wait()

        @pl.when(jnp.logical_not(is_last))
        def _():
            drain = pltpu.make_async_remote_copy(
                src_ref=send_ref,
                dst_ref=halo_ref,
                send_sem=send_sem,
                recv_sem=recv_sem,
                device_id=(right,),
                device_id_type=pl.DeviceIdType.MESH,
            )
            drain.wait_send()

    return pl.pallas_call(
        body,
        out_shape=jax.ShapeDtypeStruct((b, s, c), jnp.bfloat16),
        in_specs=[
            pl.BlockSpec(memory_space=pl.ANY),
            pl.BlockSpec(memory_space=pl.ANY),
        ],
        out_specs=pl.BlockSpec(memory_space=pl.ANY),
        scratch_shapes=[
            pltpu.VMEM((b, s, c), x.dtype),
            pltpu.VMEM((taps, c), k.dtype),
            pltpu.VMEM((b, s, c), jnp.bfloat16),
            pltpu.VMEM((b, halo, c), x.dtype),
            pltpu.VMEM((b, halo, c), x.dtype),
            pltpu.SemaphoreType.DMA((2,)),
            pltpu.SemaphoreType.DMA((2,)),
            pltpu.SemaphoreType.DMA,
            pltpu.SemaphoreType.DMA,
        ],
        compiler_params=pltpu.CompilerParams(collective_id=0),
    )(x, k)
